# Initial kernel scaffold; baseline (speedup 1.0000x reference)
import jax
import jax.numpy as jnp
from jax.experimental import pallas as pl


def kernel(node_feat, coord_feat, edge_index, edge_feat, We1, be1, We2, be2, Wn1, bn1, Wn2, bn2, Wc1, bc1, Wc2):
    raise NotImplementedError("write your pallas kernel here")


# speedup vs baseline: 3.4841x; 3.4841x over previous
import functools

import jax
import jax.numpy as jnp
from jax import lax
from jax.experimental import pallas as pl
from jax.experimental.pallas import tpu as pltpu
from jax.experimental.pallas import tpu_sc as plsc

N = 10000
E = 320000
D = 128
DE = 16

NC = 2
NS = 16
NW = NC * NS
EPW = E // NW
GB = 80
NB = EPW // GB
NP = 10240
RPT = NP // NS
RC = GB
NRC = RPT // RC

_SC_MESH = dict(core_axis_name="c", subcore_axis_name="s",
                num_cores=NC, num_subcores=NS)


def _pre_body(nf_ref, ws_ref, wd_ref, ps_ref, pd_ref):
    x = nf_ref[...]
    ps_ref[...] = jnp.dot(x, ws_ref[...], preferred_element_type=jnp.float32)
    pd_ref[...] = jnp.dot(x, wd_ref[...], preferred_element_type=jnp.float32)


_BN1 = 2000

_pre_call = pl.pallas_call(
    _pre_body,
    grid=(N // _BN1,),
    in_specs=[
        pl.BlockSpec((_BN1, D), lambda i: (i, 0)),
        pl.BlockSpec((D, D), lambda i: (0, 0)),
        pl.BlockSpec((D, D), lambda i: (0, 0)),
    ],
    out_specs=[
        pl.BlockSpec((_BN1, D), lambda i: (i, 0)),
        pl.BlockSpec((_BN1, D), lambda i: (i, 0)),
    ],
    out_shape=[jax.ShapeDtypeStruct((N, D), jnp.float32)] * 2,
)


def _gather_body(ps_hbm, pd_hbm, coord_hbm, sidx_hbm, didx_hbm,
                 u_hbm, g_hbm,
                 bufa, bufb, coordv, sblk, dblk, gbuf):
    c = lax.axis_index("c")
    s = lax.axis_index("s")
    wid = c * NS + s
    pltpu.sync_copy(coord_hbm, coordv)

    def block(b, carry):
        ebase = wid * EPW + b * GB
        pltpu.sync_copy(sidx_hbm.at[wid, b], sblk)
        pltpu.sync_copy(didx_hbm.at[wid, b], dblk)
        pltpu.sync_copy(ps_hbm.at[sblk], bufa)
        pltpu.sync_copy(pd_hbm.at[dblk], bufb)

        def addrow(r, cr):
            for cc in range(D // 16):
                sl = pl.ds(cc * 16, 16)
                bufa[r, sl] = bufa[r, sl] + bufb[r, sl]
            return cr

        lax.fori_loop(0, GB, addrow, 0)

        def coords(j, cr):
            sl = pl.ds(j * 16, 16)
            sv = sblk[sl]
            dv = dblk[sl]
            rowi = j * 16 + lax.iota(jnp.int32, 16)
            rad = jnp.zeros((16,), jnp.float32)
            for comp in range(3):
                cvec = jnp.full((16,), comp, jnp.int32)
                a = plsc.load_gather(coordv, [cvec, sv])
                bb = plsc.load_gather(coordv, [cvec, dv])
                dlt = a - bb
                plsc.store_scatter(gbuf, [rowi, cvec], dlt)
                rad = rad + dlt * dlt
            plsc.store_scatter(gbuf, [rowi, jnp.full((16,), 3, jnp.int32)], rad)
            return cr

        lax.fori_loop(0, GB // 16, coords, 0)
        pltpu.sync_copy(bufa, u_hbm.at[pl.ds(ebase, GB)])
        pltpu.sync_copy(gbuf, g_hbm.at[pl.ds(ebase, GB)])
        return carry

    lax.fori_loop(0, NB, block, 0)


@functools.lru_cache(maxsize=None)
def _gather_call():
    return pl.kernel(
        _gather_body,
        out_type=[
            jax.ShapeDtypeStruct((E, D), jnp.float32),
            jax.ShapeDtypeStruct((E, 8), jnp.float32),
        ],
        mesh=plsc.VectorSubcoreMesh(**_SC_MESH),
        scratch_types=[
            pltpu.VMEM((GB, D), jnp.float32),
            pltpu.VMEM((GB, D), jnp.float32),
            pltpu.VMEM((3, N), jnp.float32),
            pltpu.VMEM((GB,), jnp.int32),
            pltpu.VMEM((GB,), jnp.int32),
            pltpu.VMEM((GB, 8), jnp.float32),
        ],
        compiler_params=pltpu.CompilerParams(needs_layout_passes=False),
    )


def _edge_body(u_ref, ef_ref, g_ref, wee_ref, wr_ref, be1_ref, we2_ref,
               be2_ref, wc1_ref, bc1_ref, wc2_ref, mh_ref, mx_ref):
    g = g_ref[...]
    rad = g[:, 3:4]
    u1 = (u_ref[...] + rad * wr_ref[...]
          + jnp.dot(ef_ref[...], wee_ref[...],
                    preferred_element_type=jnp.float32)
          + be1_ref[...])
    h1 = u1 * jax.nn.sigmoid(u1)
    t2 = jnp.dot(h1, we2_ref[...], preferred_element_type=jnp.float32) \
        + be2_ref[...]
    mh = t2 * jax.nn.sigmoid(t2)
    mh_ref[...] = mh
    t3 = jnp.dot(mh, wc1_ref[...], preferred_element_type=jnp.float32) \
        + bc1_ref[...]
    c1 = t3 * jax.nn.sigmoid(t3)
    coef = jnp.sum(c1 * wc2_ref[...], axis=1, keepdims=True)
    scale = coef / (jnp.sqrt(rad) + 1e-30)
    lane = lax.broadcasted_iota(jnp.int32, (1, 8), 1)
    mx_ref[...] = jnp.where(lane == 3, 1.0, g * scale)


_BE = 2000

_edge_call = pl.pallas_call(
    _edge_body,
    grid=(E // _BE,),
    in_specs=[
        pl.BlockSpec((_BE, D), lambda i: (i, 0)),
        pl.BlockSpec((_BE, DE), lambda i: (i, 0)),
        pl.BlockSpec((_BE, 8), lambda i: (i, 0)),
        pl.BlockSpec((DE, D), lambda i: (0, 0)),
        pl.BlockSpec((1, D), lambda i: (0, 0)),
        pl.BlockSpec((1, D), lambda i: (0, 0)),
        pl.BlockSpec((D, D), lambda i: (0, 0)),
        pl.BlockSpec((1, D), lambda i: (0, 0)),
        pl.BlockSpec((D, D), lambda i: (0, 0)),
        pl.BlockSpec((1, D), lambda i: (0, 0)),
        pl.BlockSpec((1, D), lambda i: (0, 0)),
    ],
    out_specs=[
        pl.BlockSpec((_BE, D), lambda i: (i, 0)),
        pl.BlockSpec((_BE, 8), lambda i: (i, 0)),
    ],
    out_shape=[
        jax.ShapeDtypeStruct((E, D), jnp.float32),
        jax.ShapeDtypeStruct((E, 8), jnp.float32),
    ],
)


def _scatter_body(mh_hbm, mx_hbm, didx_hbm, z128_hbm, z8_hbm,
                  hp_hbm, xp_hbm,
                  hacc, xacc, mhbuf, mxbuf, dblk, ridx):
    c = lax.axis_index("c")
    s = lax.axis_index("s")
    wid = c * NS + s
    r0 = s * RPT

    def fill_ridx(base):
        def w(j, cr):
            ridx[pl.ds(j * 16, 16)] = base + j * 16 + lax.iota(jnp.int32, 16)
            return cr
        lax.fori_loop(0, GB // 16, w, 0)

    pltpu.sync_copy(z128_hbm, mhbuf)
    pltpu.sync_copy(z8_hbm, mxbuf)
    for k in range(NRC):
        fill_ridx(r0 + k * RC)
        pltpu.sync_copy(mhbuf, hacc.at[ridx])
        pltpu.sync_copy(mxbuf, xacc.at[ridx])
    plsc.subcore_barrier()

    def block(b, carry):
        ebase = wid * EPW + b * GB
        pltpu.sync_copy(didx_hbm.at[wid, b], dblk)
        pltpu.sync_copy(mh_hbm.at[pl.ds(ebase, GB)], mhbuf)
        pltpu.sync_copy(mx_hbm.at[pl.ds(ebase, GB)], mxbuf)
        pltpu.sync_copy(mhbuf, hacc.at[dblk], add=True)
        pltpu.sync_copy(mxbuf, xacc.at[dblk], add=True)
        return carry

    lax.fori_loop(0, NB, block, 0)
    plsc.subcore_barrier()
    for k in range(NRC):
        fill_ridx(r0 + k * RC)
        pltpu.sync_copy(hacc.at[ridx], mhbuf)
        pltpu.sync_copy(xacc.at[ridx], mxbuf)
        pltpu.sync_copy(mhbuf, hp_hbm.at[c, pl.ds(r0 + k * RC, RC)])
        pltpu.sync_copy(mxbuf, xp_hbm.at[c, pl.ds(r0 + k * RC, RC)])


@functools.lru_cache(maxsize=None)
def _scatter_call():
    return pl.kernel(
        _scatter_body,
        out_type=[
            jax.ShapeDtypeStruct((NC, NP, D), jnp.float32),
            jax.ShapeDtypeStruct((NC, NP, 8), jnp.float32),
        ],
        mesh=plsc.VectorSubcoreMesh(**_SC_MESH),
        scratch_types=[
            pltpu.VMEM_SHARED((NP, D), jnp.float32),
            pltpu.VMEM_SHARED((NP, 8), jnp.float32),
            pltpu.VMEM((GB, D), jnp.float32),
            pltpu.VMEM((GB, 8), jnp.float32),
            pltpu.VMEM((GB,), jnp.int32),
            pltpu.VMEM((GB,), jnp.int32),
        ],
        compiler_params=pltpu.CompilerParams(needs_layout_passes=False),
    )


def _node_body(nf_ref, cf_ref, hp_ref, xp_ref, wn1n_ref, wn1h_ref, bn1_ref,
               wn2_ref, bn2_ref, h_ref, x_ref):
    hn = hp_ref[0] + hp_ref[1]
    t = (jnp.dot(nf_ref[...], wn1n_ref[...],
                 preferred_element_type=jnp.float32)
         + jnp.dot(hn, wn1h_ref[...], preferred_element_type=jnp.float32)
         + bn1_ref[...])
    t = t * jax.nn.sigmoid(t)
    h_ref[...] = jnp.dot(t, wn2_ref[...],
                         preferred_element_type=jnp.float32) + bn2_ref[...]
    x8 = xp_ref[0] + xp_ref[1]
    deg = jnp.maximum(x8[:, 3:4], 1.0)
    x_ref[...] = cf_ref[...] + x8[:, 0:3] / deg


_BN2 = 2000

_node_call = pl.pallas_call(
    _node_body,
    grid=(N // _BN2,),
    in_specs=[
        pl.BlockSpec((_BN2, D), lambda i: (i, 0)),
        pl.BlockSpec((_BN2, 3), lambda i: (i, 0)),
        pl.BlockSpec((NC, _BN2, D), lambda i: (0, i, 0)),
        pl.BlockSpec((NC, _BN2, 8), lambda i: (0, i, 0)),
        pl.BlockSpec((D, D), lambda i: (0, 0)),
        pl.BlockSpec((D, D), lambda i: (0, 0)),
        pl.BlockSpec((1, D), lambda i: (0, 0)),
        pl.BlockSpec((D, D), lambda i: (0, 0)),
        pl.BlockSpec((1, D), lambda i: (0, 0)),
    ],
    out_specs=[
        pl.BlockSpec((_BN2, D), lambda i: (i, 0)),
        pl.BlockSpec((_BN2, 3), lambda i: (i, 0)),
    ],
    out_shape=[
        jax.ShapeDtypeStruct((N, D), jnp.float32),
        jax.ShapeDtypeStruct((N, 3), jnp.float32),
    ],
)


def kernel(node_feat, coord_feat, edge_index, edge_feat,
           We1, be1, We2, be2, Wn1, bn1, Wn2, bn2, Wc1, bc1, Wc2):
    src = edge_index[0].reshape(NW, NB, GB)
    dst = edge_index[1].reshape(NW, NB, GB)
    coord_t = coord_feat.T
    we1_s = We1[:D]
    we1_d = We1[D:2 * D]
    we1_r = We1[2 * D:2 * D + 1]
    we1_e = We1[2 * D + 1:]

    ps, pd = _pre_call(node_feat, we1_s, we1_d)
    u, g = _gather_call()(ps, pd, coord_t, src, dst)
    mh, mx = _edge_call(u, edge_feat, g, we1_e, we1_r,
                        be1.reshape(1, D), We2, be2.reshape(1, D),
                        Wc1, bc1.reshape(1, D), Wc2.reshape(1, D))
    z128 = jnp.zeros((RC, D), jnp.float32)
    z8 = jnp.zeros((RC, 8), jnp.float32)
    hp, xp = _scatter_call()(mh, mx, dst, z128, z8)
    h_out, x_out = _node_call(node_feat, coord_feat, hp, xp,
                              Wn1[:D], Wn1[D:], bn1.reshape(1, D),
                              Wn2, bn2.reshape(1, D))
    return (h_out, x_out)


# speedup vs baseline: 5.3285x; 1.5294x over previous
import functools

import jax
import jax.numpy as jnp
from jax import lax
from jax.experimental import pallas as pl
from jax.experimental.pallas import tpu as pltpu
from jax.experimental.pallas import tpu_sc as plsc

N = 10000
E = 320000
D = 128
DE = 16

NC = 2
NS = 16
NW = NC * NS
EPW = E // NW
GB = 80
NB = EPW // GB
NP = 10240
RPT = NP // NS
RC = GB
NRC = RPT // RC

_SC_MESH = dict(core_axis_name="c", subcore_axis_name="s",
                num_cores=NC, num_subcores=NS)


def _pre_body(nf_ref, ws_ref, wd_ref, ps_ref, pd_ref):
    x = nf_ref[...]
    ps_ref[...] = jnp.dot(x, ws_ref[...], preferred_element_type=jnp.float32)
    pd_ref[...] = jnp.dot(x, wd_ref[...], preferred_element_type=jnp.float32)


_BN1 = 2000

_pre_call = pl.pallas_call(
    _pre_body,
    grid=(N // _BN1,),
    in_specs=[
        pl.BlockSpec((_BN1, D), lambda i: (i, 0)),
        pl.BlockSpec((D, D), lambda i: (0, 0)),
        pl.BlockSpec((D, D), lambda i: (0, 0)),
    ],
    out_specs=[
        pl.BlockSpec((_BN1, D), lambda i: (i, 0)),
        pl.BlockSpec((_BN1, D), lambda i: (i, 0)),
    ],
    out_shape=[jax.ShapeDtypeStruct((N, D), jnp.float32)] * 2,
)


_NSUP = NB // 2


def _gather_body(ps_hbm, pd_hbm, coord_hbm, sidx_hbm, didx_hbm,
                 u_hbm, g_hbm,
                 bufa0, bufa1, bufb0, bufb1, gbuf0, gbuf1, coordv,
                 sblk0, sblk1, dblk0, dblk1,
                 semi0, semi1, sema0, sema1, semb0, semb1,
                 semu0, semu1, semg0, semg1):
    c = lax.axis_index("c")
    s = lax.axis_index("s")
    wid = c * NS + s
    bufa = (bufa0, bufa1)
    bufb = (bufb0, bufb1)
    gbuf = (gbuf0, gbuf1)
    sblk = (sblk0, sblk1)
    dblk = (dblk0, dblk1)
    semi = (semi0, semi1)
    sema = (sema0, sema1)
    semb = (semb0, semb1)
    semu = (semu0, semu1)
    semg = (semg0, semg1)

    pltpu.sync_copy(coord_hbm, coordv)

    def start_idx(p, b):
        pltpu.async_copy(sidx_hbm.at[wid, b], sblk[p], semi[p])
        pltpu.async_copy(didx_hbm.at[wid, b], dblk[p], semi[p])

    def wait_idx(p):
        pltpu.make_async_copy(sidx_hbm.at[wid, 0], sblk[p], semi[p]).wait()
        pltpu.make_async_copy(didx_hbm.at[wid, 0], dblk[p], semi[p]).wait()

    def start_in(p):
        pltpu.async_copy(ps_hbm.at[sblk[p]], bufa[p], sema[p])
        pltpu.async_copy(pd_hbm.at[dblk[p]], bufb[p], semb[p])

    def wait_in(p):
        pltpu.make_async_copy(ps_hbm.at[sblk[p]], bufa[p], sema[p]).wait()
        pltpu.make_async_copy(pd_hbm.at[dblk[p]], bufb[p], semb[p]).wait()

    def start_out(p, b):
        ebase = wid * EPW + b * GB
        pltpu.async_copy(bufa[p], u_hbm.at[pl.ds(ebase, GB)], semu[p])
        pltpu.async_copy(gbuf[p], g_hbm.at[pl.ds(ebase, GB)], semg[p])

    def wait_out(p):
        pltpu.make_async_copy(bufa[p], u_hbm.at[pl.ds(0, GB)], semu[p]).wait()
        pltpu.make_async_copy(gbuf[p], g_hbm.at[pl.ds(0, GB)], semg[p]).wait()

    def process(p):
        ba, bb, gb, sb, db = bufa[p], bufb[p], gbuf[p], sblk[p], dblk[p]

        def addrow(r, cr):
            for cc in range(D // 16):
                sl = pl.ds(cc * 16, 16)
                ba[r, sl] = ba[r, sl] + bb[r, sl]
            return cr

        lax.fori_loop(0, GB, addrow, 0)

        def coords(j, cr):
            sl = pl.ds(j * 16, 16)
            sv = sb[sl]
            dv = db[sl]
            rowi = j * 16 + lax.iota(jnp.int32, 16)
            rad = jnp.zeros((16,), jnp.float32)
            for comp in range(3):
                cvec = jnp.full((16,), comp, jnp.int32)
                a = plsc.load_gather(coordv, [cvec, sv])
                bb2 = plsc.load_gather(coordv, [cvec, dv])
                dlt = a - bb2
                plsc.store_scatter(gb, [rowi, cvec], dlt)
                rad = rad + dlt * dlt
            plsc.store_scatter(gb, [rowi, jnp.full((16,), 3, jnp.int32)],
                               rad)
            return cr

        lax.fori_loop(0, GB // 16, coords, 0)

    start_idx(0, 0)
    wait_idx(0)
    start_in(0)
    start_idx(1, 1)

    def super_block(i, carry):
        b0 = 2 * i
        b1 = b0 + 1
        wait_idx(1)
        start_in(1)
        wait_in(0)
        process(0)
        start_out(0, b0)
        wait_out(0)
        start_idx(0, b0 + 2)
        wait_idx(0)
        start_in(0)
        wait_in(1)
        process(1)
        start_out(1, b1)
        wait_out(1)

        @pl.when(i < _NSUP - 1)
        def _():
            start_idx(1, b1 + 2)
        return carry

    lax.fori_loop(0, _NSUP, super_block, 0)
    wait_in(0)
    process(0)
    start_out(0, NB - 1)
    wait_out(0)


@functools.lru_cache(maxsize=None)
def _gather_call():
    return pl.kernel(
        _gather_body,
        out_type=[
            jax.ShapeDtypeStruct((E, D), jnp.float32),
            jax.ShapeDtypeStruct((E, 8), jnp.float32),
        ],
        mesh=plsc.VectorSubcoreMesh(**_SC_MESH),
        scratch_types=[
            pltpu.VMEM((GB, D), jnp.float32),
            pltpu.VMEM((GB, D), jnp.float32),
            pltpu.VMEM((GB, D), jnp.float32),
            pltpu.VMEM((GB, D), jnp.float32),
            pltpu.VMEM((GB, 8), jnp.float32),
            pltpu.VMEM((GB, 8), jnp.float32),
            pltpu.VMEM((3, N), jnp.float32),
            pltpu.VMEM((GB,), jnp.int32),
            pltpu.VMEM((GB,), jnp.int32),
            pltpu.VMEM((GB,), jnp.int32),
            pltpu.VMEM((GB,), jnp.int32),
        ] + [pltpu.SemaphoreType.DMA] * 10,
        compiler_params=pltpu.CompilerParams(needs_layout_passes=False),
    )


def _edge_body(u_ref, ef_ref, g_ref, wee_ref, wr_ref, be1_ref, we2_ref,
               be2_ref, wc1_ref, bc1_ref, wc2_ref, mh_ref, mx_ref):
    g = g_ref[...]
    rad = g[:, 3:4]
    u1 = (u_ref[...] + rad * wr_ref[...]
          + jnp.dot(ef_ref[...], wee_ref[...],
                    preferred_element_type=jnp.float32)
          + be1_ref[...])
    h1 = u1 * jax.nn.sigmoid(u1)
    t2 = jnp.dot(h1, we2_ref[...], preferred_element_type=jnp.float32) \
        + be2_ref[...]
    mh = t2 * jax.nn.sigmoid(t2)
    mh_ref[...] = mh
    t3 = jnp.dot(mh, wc1_ref[...], preferred_element_type=jnp.float32) \
        + bc1_ref[...]
    c1 = t3 * jax.nn.sigmoid(t3)
    coef = jnp.sum(c1 * wc2_ref[...], axis=1, keepdims=True)
    scale = coef / (jnp.sqrt(rad) + 1e-30)
    lane = lax.broadcasted_iota(jnp.int32, (1, 8), 1)
    mx_ref[...] = jnp.where(lane == 3, 1.0, g * scale)


_BE = 2000

_edge_call = pl.pallas_call(
    _edge_body,
    grid=(E // _BE,),
    in_specs=[
        pl.BlockSpec((_BE, D), lambda i: (i, 0)),
        pl.BlockSpec((_BE, DE), lambda i: (i, 0)),
        pl.BlockSpec((_BE, 8), lambda i: (i, 0)),
        pl.BlockSpec((DE, D), lambda i: (0, 0)),
        pl.BlockSpec((1, D), lambda i: (0, 0)),
        pl.BlockSpec((1, D), lambda i: (0, 0)),
        pl.BlockSpec((D, D), lambda i: (0, 0)),
        pl.BlockSpec((1, D), lambda i: (0, 0)),
        pl.BlockSpec((D, D), lambda i: (0, 0)),
        pl.BlockSpec((1, D), lambda i: (0, 0)),
        pl.BlockSpec((1, D), lambda i: (0, 0)),
    ],
    out_specs=[
        pl.BlockSpec((_BE, D), lambda i: (i, 0)),
        pl.BlockSpec((_BE, 8), lambda i: (i, 0)),
    ],
    out_shape=[
        jax.ShapeDtypeStruct((E, D), jnp.float32),
        jax.ShapeDtypeStruct((E, 8), jnp.float32),
    ],
)


def _scatter_body(mh_hbm, mx_hbm, didx_hbm, z128_hbm, z8_hbm,
                  hp_hbm, xp_hbm,
                  hacc, xacc, mhb0, mhb1, mxb0, mxb1, dblk0, dblk1, ridx,
                  semi0, semi1, semh0, semh1, semx0, semx1):
    c = lax.axis_index("c")
    s = lax.axis_index("s")
    wid = c * NS + s
    r0 = s * RPT
    mhb = (mhb0, mhb1)
    mxb = (mxb0, mxb1)
    dblk = (dblk0, dblk1)
    semi = (semi0, semi1)
    semh = (semh0, semh1)
    semx = (semx0, semx1)

    def fill_ridx(base):
        def w(j, cr):
            ridx[pl.ds(j * 16, 16)] = base + j * 16 + lax.iota(jnp.int32, 16)
            return cr
        lax.fori_loop(0, GB // 16, w, 0)

    pltpu.sync_copy(z128_hbm, mhb0)
    pltpu.sync_copy(z8_hbm, mxb0)
    for k in range(NRC):
        fill_ridx(r0 + k * RC)
        pltpu.sync_copy(mhb0, hacc.at[ridx])
        pltpu.sync_copy(mxb0, xacc.at[ridx])
    plsc.subcore_barrier()

    def start_in(p, b):
        ebase = wid * EPW + b * GB
        pltpu.async_copy(didx_hbm.at[wid, b], dblk[p], semi[p])
        pltpu.async_copy(mh_hbm.at[pl.ds(ebase, GB)], mhb[p], semh[p])
        pltpu.async_copy(mx_hbm.at[pl.ds(ebase, GB)], mxb[p], semx[p])

    def wait_in(p):
        pltpu.make_async_copy(didx_hbm.at[wid, 0], dblk[p], semi[p]).wait()
        pltpu.make_async_copy(mh_hbm.at[pl.ds(0, GB)], mhb[p], semh[p]).wait()
        pltpu.make_async_copy(mx_hbm.at[pl.ds(0, GB)], mxb[p], semx[p]).wait()

    def add(p):
        pltpu.sync_copy(mhb[p], hacc.at[dblk[p]], add=True)
        pltpu.sync_copy(mxb[p], xacc.at[dblk[p]], add=True)

    start_in(0, 0)

    def super_block(i, carry):
        b0 = 2 * i
        b1 = b0 + 1
        start_in(1, b1)
        wait_in(0)
        add(0)
        start_in(0, b0 + 2)
        wait_in(1)
        add(1)
        return carry

    lax.fori_loop(0, _NSUP, super_block, 0)
    wait_in(0)
    add(0)
    plsc.subcore_barrier()
    for k in range(NRC):
        fill_ridx(r0 + k * RC)
        pltpu.sync_copy(hacc.at[ridx], mhb0)
        pltpu.sync_copy(xacc.at[ridx], mxb0)
        pltpu.sync_copy(mhb0, hp_hbm.at[c, pl.ds(r0 + k * RC, RC)])
        pltpu.sync_copy(mxb0, xp_hbm.at[c, pl.ds(r0 + k * RC, RC)])


@functools.lru_cache(maxsize=None)
def _scatter_call():
    return pl.kernel(
        _scatter_body,
        out_type=[
            jax.ShapeDtypeStruct((NC, NP, D), jnp.float32),
            jax.ShapeDtypeStruct((NC, NP, 8), jnp.float32),
        ],
        mesh=plsc.VectorSubcoreMesh(**_SC_MESH),
        scratch_types=[
            pltpu.VMEM_SHARED((NP, D), jnp.float32),
            pltpu.VMEM_SHARED((NP, 8), jnp.float32),
            pltpu.VMEM((GB, D), jnp.float32),
            pltpu.VMEM((GB, D), jnp.float32),
            pltpu.VMEM((GB, 8), jnp.float32),
            pltpu.VMEM((GB, 8), jnp.float32),
            pltpu.VMEM((GB,), jnp.int32),
            pltpu.VMEM((GB,), jnp.int32),
            pltpu.VMEM((GB,), jnp.int32),
        ] + [pltpu.SemaphoreType.DMA] * 6,
        compiler_params=pltpu.CompilerParams(needs_layout_passes=False),
    )


def _node_body(nf_ref, cf_ref, hp_ref, xp_ref, wn1n_ref, wn1h_ref, bn1_ref,
               wn2_ref, bn2_ref, h_ref, x_ref):
    hn = hp_ref[0] + hp_ref[1]
    t = (jnp.dot(nf_ref[...], wn1n_ref[...],
                 preferred_element_type=jnp.float32)
         + jnp.dot(hn, wn1h_ref[...], preferred_element_type=jnp.float32)
         + bn1_ref[...])
    t = t * jax.nn.sigmoid(t)
    h_ref[...] = jnp.dot(t, wn2_ref[...],
                         preferred_element_type=jnp.float32) + bn2_ref[...]
    x8 = xp_ref[0] + xp_ref[1]
    deg = jnp.maximum(x8[:, 3:4], 1.0)
    x_ref[...] = cf_ref[...] + x8[:, 0:3] / deg


_BN2 = 2000

_node_call = pl.pallas_call(
    _node_body,
    grid=(N // _BN2,),
    in_specs=[
        pl.BlockSpec((_BN2, D), lambda i: (i, 0)),
        pl.BlockSpec((_BN2, 3), lambda i: (i, 0)),
        pl.BlockSpec((NC, _BN2, D), lambda i: (0, i, 0)),
        pl.BlockSpec((NC, _BN2, 8), lambda i: (0, i, 0)),
        pl.BlockSpec((D, D), lambda i: (0, 0)),
        pl.BlockSpec((D, D), lambda i: (0, 0)),
        pl.BlockSpec((1, D), lambda i: (0, 0)),
        pl.BlockSpec((D, D), lambda i: (0, 0)),
        pl.BlockSpec((1, D), lambda i: (0, 0)),
    ],
    out_specs=[
        pl.BlockSpec((_BN2, D), lambda i: (i, 0)),
        pl.BlockSpec((_BN2, 3), lambda i: (i, 0)),
    ],
    out_shape=[
        jax.ShapeDtypeStruct((N, D), jnp.float32),
        jax.ShapeDtypeStruct((N, 3), jnp.float32),
    ],
)


def kernel(node_feat, coord_feat, edge_index, edge_feat,
           We1, be1, We2, be2, Wn1, bn1, Wn2, bn2, Wc1, bc1, Wc2):
    src = edge_index[0].reshape(NW, NB, GB)
    dst = edge_index[1].reshape(NW, NB, GB)
    coord_t = coord_feat.T
    we1_s = We1[:D]
    we1_d = We1[D:2 * D]
    we1_r = We1[2 * D:2 * D + 1]
    we1_e = We1[2 * D + 1:]

    ps, pd = _pre_call(node_feat, we1_s, we1_d)
    u, g = _gather_call()(ps, pd, coord_t, src, dst)
    mh, mx = _edge_call(u, edge_feat, g, we1_e, we1_r,
                        be1.reshape(1, D), We2, be2.reshape(1, D),
                        Wc1, bc1.reshape(1, D), Wc2.reshape(1, D))
    z128 = jnp.zeros((RC, D), jnp.float32)
    z8 = jnp.zeros((RC, 8), jnp.float32)
    hp, xp = _scatter_call()(mh, mx, dst, z128, z8)
    h_out, x_out = _node_call(node_feat, coord_feat, hp, xp,
                              Wn1[:D], Wn1[D:], bn1.reshape(1, D),
                              Wn2, bn2.reshape(1, D))
    return (h_out, x_out)
